# baseline (speedup 1.0000x reference)
import functools

import jax
import jax.numpy as jnp
from jax import lax
from jax.experimental import pallas as pl
from jax.experimental.pallas import tpu as pltpu
from jax.experimental.pallas import tpu_sc as plsc

_B, _S, _D = 4, 2048, 1024
_NC, _NS = 2, 16
_NW = _NC * _NS
_RPW = (_B * _S) // _NW
_C = 32
_NCHUNK = _RPW // _C

_mesh = plsc.VectorSubcoreMesh(core_axis_name="c", subcore_axis_name="s")


@functools.partial(
    pl.kernel,
    mesh=_mesh,
    out_type=jax.ShapeDtypeStruct((_B * _S, _D), jnp.float32),
    scratch_types=[
        pltpu.VMEM((_RPW,), jnp.int32),
        pltpu.VMEM((2, _C, _D), jnp.float32),
        pltpu.VMEM((_C, _D), jnp.float32),
        pltpu.SemaphoreType.DMA,
        pltpu.SemaphoreType.DMA,
        pltpu.SemaphoreType.DMA,
        pltpu.SemaphoreType.DMA,
    ],
)
def _embed(x_hbm, tok_hbm, pos_hbm, out_hbm, idx_v, tok_v, pos_v,
           gsem0, gsem1, wsem0, wsem1):
    cid = lax.axis_index("c")
    sid = lax.axis_index("s")
    wid = sid * _NC + cid
    base = wid * _RPW
    s_base = lax.rem(base, _S)

    gsems = [gsem0, gsem1]
    wsems = [wsem0, wsem1]

    pltpu.sync_copy(x_hbm.at[pl.ds(base, _RPW)], idx_v)

    gathers = [None] * _NCHUNK
    writes = [None] * _NCHUNK

    def start_gather(i):
        buf = i % 2
        gathers[i] = pltpu.async_copy(
            tok_hbm.at[idx_v.at[pl.ds(i * _C, _C)]], tok_v.at[buf],
            gsems[buf])

    start_gather(0)
    for i in range(_NCHUNK):
        buf = i % 2
        if i + 1 < _NCHUNK:
            if i >= 1:
                writes[i - 1].wait()
            start_gather(i + 1)
        gathers[i].wait()
        pltpu.sync_copy(pos_hbm.at[pl.ds(s_base + i * _C, _C)], pos_v)

        def row_add(r, _):
            for j in range(_D // 16):
                vec = pos_v[r, pl.ds(j * 16, 16)]
                plsc.addupdate(tok_v.at[buf, r, pl.ds(j * 16, 16)], vec)
            return 0

        lax.fori_loop(0, _C, row_add, 0)

        writes[i] = pltpu.async_copy(
            tok_v.at[buf], out_hbm.at[pl.ds(base + i * _C, _C)], wsems[buf])

    writes[_NCHUNK - 2].wait()
    writes[_NCHUNK - 1].wait()


@jax.jit
def kernel(x, token_table, pos_table):
    xf = x.reshape(-1).astype(jnp.int32)
    out = _embed(xf, token_table, pos_table)
    return out.reshape(_B, _S, _D)


# speedup vs baseline: 1.6479x; 1.6479x over previous
import functools

import jax
import jax.numpy as jnp
from jax import lax
from jax.experimental import pallas as pl
from jax.experimental.pallas import tpu as pltpu
from jax.experimental.pallas import tpu_sc as plsc

_B, _S, _D = 4, 2048, 1024
_NC, _NS = 2, 16
_NW = _NC * _NS
_SPW = _S // _NW
_RPW = _B * _SPW
_CS = 4
_C = _B * _CS
_NCHUNK = _SPW // _CS
_NBUF = 4
_NPOS = 2

_mesh = plsc.VectorSubcoreMesh(core_axis_name="c", subcore_axis_name="s")


@functools.partial(
    pl.kernel,
    mesh=_mesh,
    out_type=jax.ShapeDtypeStruct((_B * _S, _D), jnp.float32),
    scratch_types=[
        pltpu.VMEM((_RPW,), jnp.int32),
        pltpu.VMEM((_NBUF, _C, _D), jnp.float32),
        pltpu.VMEM((_NPOS, _CS, _D), jnp.float32),
        pltpu.SemaphoreType.DMA,
        pltpu.SemaphoreType.DMA,
        pltpu.SemaphoreType.DMA,
        pltpu.SemaphoreType.DMA,
        pltpu.SemaphoreType.DMA,
        pltpu.SemaphoreType.DMA,
        pltpu.SemaphoreType.DMA,
        pltpu.SemaphoreType.DMA,
        pltpu.SemaphoreType.DMA,
        pltpu.SemaphoreType.DMA,
    ],
)
def _embed(xt_hbm, tok_hbm, pos_hbm, out_hbm, idx_v, tok_v, pos_v,
           g0, g1, g2, g3, w0, w1, w2, w3, p0, p1):
    cid = lax.axis_index("c")
    sid = lax.axis_index("s")
    wid = sid * _NC + cid
    s0 = wid * _SPW

    gsems = [g0, g1, g2, g3]
    wsems = [w0, w1, w2, w3]
    psems = [p0, p1]

    pltpu.sync_copy(xt_hbm.at[pl.ds(wid * _RPW, _RPW)], idx_v)

    gathers = [None] * _NCHUNK
    writes = [None] * _NCHUNK
    poss = [None] * _NCHUNK

    def start_gather(i):
        b = i % _NBUF
        gathers[i] = pltpu.async_copy(
            tok_hbm.at[idx_v.at[pl.ds(i * _C, _C)]], tok_v.at[b], gsems[b])

    def start_pos(i):
        p = i % _NPOS
        poss[i] = pltpu.async_copy(
            pos_hbm.at[pl.ds(s0 + i * _CS, _CS)], pos_v.at[p], psems[p])

    for i in range(3):
        start_gather(i)
    for i in range(_NPOS):
        start_pos(i)

    for i in range(_NCHUNK):
        b = i % _NBUF
        p = i % _NPOS
        gathers[i].wait()
        poss[i].wait()

        def vadd_body(j, _):
            for t in range(_CS):
                vec = pos_v[p, t, pl.ds(j * 16, 16)]
                for bb in range(_B):
                    plsc.addupdate(
                        tok_v.at[b, bb * _CS + t, pl.ds(j * 16, 16)], vec)
            return 0

        lax.fori_loop(0, _D // 16, vadd_body, 0)

        if i + _NPOS < _NCHUNK:
            start_pos(i + _NPOS)

        writes[i] = [
            pltpu.async_copy(
                tok_v.at[b, pl.ds(bb * _CS, _CS)],
                out_hbm.at[pl.ds(bb * _S + s0 + i * _CS, _CS)], wsems[b])
            for bb in range(_B)
        ]
        if i + 3 < _NCHUNK:
            if i >= 1:
                for cp in writes[i - 1]:
                    cp.wait()
            start_gather(i + 3)

    for i in range(_NCHUNK - 4, _NCHUNK):
        for cp in writes[i]:
            cp.wait()


@jax.jit
def kernel(x, token_table, pos_table):
    x4 = x.astype(jnp.int32).reshape(_B, _NW, _NCHUNK, _CS)
    xt = x4.transpose(1, 2, 0, 3).reshape(-1)
    out = _embed(xt, token_table, pos_table)
    return out.reshape(_B, _S, _D)


# speedup vs baseline: 1.6968x; 1.0296x over previous
import functools

import jax
import jax.numpy as jnp
from jax import lax
from jax.experimental import pallas as pl
from jax.experimental.pallas import tpu as pltpu
from jax.experimental.pallas import tpu_sc as plsc

_B, _S, _D = 4, 2048, 1024
_NC, _NS = 2, 16
_NW = _NC * _NS
_SPW = _S // _NW
_CS = 4
_C = _B * _CS
_NCHUNK = _SPW // _CS
_NBUF = 6
_NPOS = 4
_LOOK = 4

_mesh = plsc.VectorSubcoreMesh(core_axis_name="c", subcore_axis_name="s")


@functools.partial(
    pl.kernel,
    mesh=_mesh,
    out_type=jax.ShapeDtypeStruct((_B * _S, _D), jnp.float32),
    scratch_types=[
        pltpu.VMEM((_B, _SPW), jnp.int32),
        pltpu.VMEM((_NBUF * _B, _CS, _D), jnp.float32),
        pltpu.VMEM((_NPOS, _CS, _D), jnp.float32),
        pltpu.SemaphoreType.DMA,
        pltpu.SemaphoreType.DMA,
        pltpu.SemaphoreType.DMA,
        pltpu.SemaphoreType.DMA,
        pltpu.SemaphoreType.DMA,
        pltpu.SemaphoreType.DMA,
        pltpu.SemaphoreType.DMA,
        pltpu.SemaphoreType.DMA,
        pltpu.SemaphoreType.DMA,
        pltpu.SemaphoreType.DMA,
        pltpu.SemaphoreType.DMA,
        pltpu.SemaphoreType.DMA,
        pltpu.SemaphoreType.DMA,
        pltpu.SemaphoreType.DMA,
        pltpu.SemaphoreType.DMA,
        pltpu.SemaphoreType.DMA,
    ],
)
def _embed(x_hbm, tok_hbm, pos_hbm, out_hbm, idx_v, tok_v, pos_v,
           g0, g1, g2, g3, g4, g5, w0, w1, w2, w3, w4, w5, p0, p1, p2, p3):
    cid = lax.axis_index("c")
    sid = lax.axis_index("s")
    wid = sid * _NC + cid
    s0 = wid * _SPW

    gsems = [g0, g1, g2, g3, g4, g5]
    wsems = [w0, w1, w2, w3, w4, w5]
    psems = [p0, p1, p2, p3]

    idx_cps = [
        pltpu.async_copy(x_hbm.at[bb, pl.ds(s0, _SPW)], idx_v.at[bb], p3)
        for bb in range(_B)
    ]
    for cp in idx_cps:
        cp.wait()

    gathers = [None] * _NCHUNK
    writes = [None] * _NCHUNK
    poss = [None] * _NCHUNK

    def start_gather(i):
        b = i % _NBUF
        gathers[i] = [
            pltpu.async_copy(
                tok_hbm.at[idx_v.at[bb, pl.ds(i * _CS, _CS)]],
                tok_v.at[b * _B + bb], gsems[b])
            for bb in range(_B)
        ]

    def start_pos(i):
        p = i % _NPOS
        poss[i] = pltpu.async_copy(
            pos_hbm.at[pl.ds(s0 + i * _CS, _CS)], pos_v.at[p], psems[p])

    for i in range(_LOOK):
        start_gather(i)
    for i in range(3):
        start_pos(i)

    for i in range(_NCHUNK):
        b = i % _NBUF
        p = i % _NPOS
        for cp in gathers[i]:
            cp.wait()
        poss[i].wait()

        def vadd_body(j, _):
            for t in range(_CS):
                vec = pos_v[p, t, pl.ds(j * 16, 16)]
                for bb in range(_B):
                    plsc.addupdate(
                        tok_v.at[b * _B + bb, t, pl.ds(j * 16, 16)], vec)
            return 0

        lax.fori_loop(0, _D // 16, vadd_body, 0)

        if i + 3 < _NCHUNK:
            start_pos(i + 3)

        writes[i] = [
            pltpu.async_copy(
                tok_v.at[b * _B + bb],
                out_hbm.at[pl.ds(bb * _S + s0 + i * _CS, _CS)], wsems[b])
            for bb in range(_B)
        ]
        if i + _LOOK < _NCHUNK:
            if i >= 2:
                for cp in writes[i - 2]:
                    cp.wait()
            start_gather(i + _LOOK)

    for i in range(_NCHUNK - _LOOK - 2, _NCHUNK):
        if i >= 0:
            for cp in writes[i]:
                cp.wait()


@jax.jit
def kernel(x, token_table, pos_table):
    out = _embed(x.astype(jnp.int32), token_table, pos_table)
    return out.reshape(_B, _S, _D)
